# baseline (speedup 1.0000x reference)
import jax
import jax.numpy as jnp
import numpy as np
from jax import lax
from jax.experimental import pallas as pl
from jax.experimental.pallas import tpu as pltpu

B, L, D, HID, PD = 4, 2048, 128, 512, 32
K = L // 2
H = 2
DH = D // H
JC = 1024

_f = np.float32


def _gelu_xla(a):
    one = _f(1.0)
    u = (-a) * _f(0.707106769)
    y = u * u
    t = y * _f(7.85386146e-05)
    t = t + _f(-0.000801019371)
    t = t * y + _f(0.00518832775)
    t = t * y + _f(-0.0268538129)
    t = t * y + _f(0.112835854)
    t = t * y + _f(-0.37612626)
    t = t * y + _f(1.12837911)
    small = one - u * t
    ny = -y
    ez = jnp.exp(ny)
    absu = jnp.abs(u)
    q = one / absu
    ezq = ez * q
    w = one / y
    p = w * _f(0.0232682)
    p = p + _f(-0.138703942)
    p = p * w + _f(0.368742466)
    p = p * w + _f(-0.582473278)
    p = p * w + _f(0.621000469)
    p = p * w + _f(-0.494451523)
    p = p * w + _f(0.340488)
    p = p * w + _f(-0.274112701)
    p = p * w + _f(0.563825965)
    r = w * _f(-10.477664)
    r = r + _f(12.9772)
    r = r * w + _f(-7.49551868)
    r = r * w + _f(2.92101908)
    r = r * w + _f(-1.01526523)
    r = r * w + _f(0.42184633)
    r = r * w + _f(-0.282076746)
    r = r * w + _f(0.564189494)
    poly = jnp.where(absu < _f(2.0), p, r)
    val = ezq * poly
    val = jnp.where(ny < _f(-88.7228394), _f(0.0), val)
    large = jnp.where(u < _f(0.0), _f(2.0) - val, val)
    erfc = jnp.where(absu < one, small, large)
    return (a * _f(0.5)) * erfc


def _mm(a, b):
    return jax.lax.dot_general(a, b, (((1,), (0,)), ((), ())),
                               preferred_element_type=jnp.float32)


def _mm_t(a, b):
    return jax.lax.dot_general(a, b, (((1,), (1,)), ((), ())),
                               preferred_element_type=jnp.float32)


def _mm_seq256(a, b):
    kk = a.shape[1]
    acc = _mm(a[:, 0:256], b[0:256, :])
    for i in range(1, kk // 256):
        acc = acc + _mm(a[:, i * 256:(i + 1) * 256], b[i * 256:(i + 1) * 256, :])
    return acc


def _rowsum_xla(x):
    n = x.shape[1]
    acc = x[:, 0:128]
    for c2 in range(1, n // 128):
        acc = acc + x[:, c2 * 128:(c2 + 1) * 128]
    a8 = acc[:, 0:8]
    for i in range(1, 16):
        a8 = a8 + acc[:, i * 8:(i + 1) * 8]
    a4 = a8[:, 0:4] + a8[:, 4:8]
    a2 = a4[:, 0:2] + a4[:, 2:4]
    return a2[:, 0:1] + a2[:, 1:2]


def _ln_stats(o):
    mean = _rowsum_xla(o) * _f(0.0078125)
    c = o - mean
    var = _rowsum_xla(c * c) * _f(0.0078125)
    sq = jnp.sqrt(var + _f(1e-5))
    return mean, sq


def _enc_body(inp, pos, Wq, bq, Wk, bk, Wv, bv, Wo, bo, n1g, n1b, n2g, n2b,
              f1W, f1b, f2W, f2b, p0W, p0b, p1W, p1b, p3W, p3b,
              xsel_o, psel_o):
    x0 = inp[0] + pos[0]
    bf = jnp.bfloat16
    qr = (_mm(x0, Wq[...]) + bq[...]).astype(bf).astype(jnp.float32)
    kr = (_mm(x0, Wk[...]) + bk[...]).astype(bf).astype(jnp.float32)
    v32 = _mm(x0, Wv[...]) + bv[...]
    outs = []
    for h in range(H):
        qh = qr[:, h * DH:(h + 1) * DH]
        kh = kr[:, h * DH:(h + 1) * DH]
        vh = v32[:, h * DH:(h + 1) * DH]
        t0 = _mm_t(qh, kh[0:JC, :]) * _f(0.125)
        m1 = jnp.max(t0, axis=1, keepdims=True)
        e0 = jnp.exp(t0 - m1)
        l1 = jnp.sum(e0, axis=1, keepdims=True)
        r0 = _mm_seq256(e0, vh[0:JC, :])
        o1 = r0 * (_f(1.0) / l1)
        t1 = _mm_t(qh, kh[JC:L, :]) * _f(0.125)
        mt = jnp.max(t1, axis=1, keepdims=True)
        m2 = jnp.maximum(m1, mt)
        corr = jnp.where(m1 == m2, _f(0.0), m1 - m2)
        c = jnp.exp(corr) * l1
        e1 = jnp.exp(t1 - m2)
        l2 = c + jnp.sum(e1, axis=1, keepdims=True)
        r1 = c * o1
        for i2 in range(JC // 256):
            r1 = r1 + _mm(e1[:, i2 * 256:(i2 + 1) * 256],
                          vh[JC + i2 * 256:JC + (i2 + 1) * 256, :])
        outs.append(r1 * (_f(1.0) / l2))
    o = jnp.concatenate(outs, axis=1)
    obf = o.astype(bf).astype(jnp.float32)
    oc = _mm(obf, Wo[...])
    ln1_in = oc + bo[...]
    mean1, sq1 = _ln_stats(ln1_in)
    ln1 = ((ln1_in - mean1) / sq1) * n1g[...] + n1b[...]
    x1 = x0 + ln1
    a1 = _mm(x1, f1W[...]) + f1b[...]
    g1 = _gelu_xla(a1)
    ln2_in = _mm(g1, f2W[...]) + f2b[...]
    mean2, sq2 = _ln_stats(ln2_in)
    ln2 = ((ln2_in - mean2) / sq2) * n2g[...] + n2b[...]
    x2 = (x0 + ln2) + ln1
    a_s0 = _mm(x2, p0W[...]) + p0b[...]
    g_s0 = _gelu_xla(a_s0)
    a_s1 = _mm(g_s0, p1W[...]) + p1b[...]
    g_s1 = _gelu_xla(a_s1)
    s_row = lax.dot_general(p3W[...], g_s1, (((0,), (1,)), ((), ())),
                            preferred_element_type=jnp.float32)
    sb_row = s_row + p3b[...]
    m_s = jnp.max(sb_row, axis=1, keepdims=True)
    e_row = jnp.exp(sb_row - m_s)
    z = _rowsum_xla(e_row)
    p_row = e_row / z
    ic = lax.broadcasted_iota(jnp.int32, (L, 1), 0)
    jr = lax.broadcasted_iota(jnp.int32, (1, L), 1)
    eye = (ic == jr).astype(jnp.float32)
    p_col = lax.dot_general(eye, p_row, (((1,), (1,)), ((), ())),
                            precision=lax.Precision.HIGHEST)
    gt = p_row > p_col
    tie = (p_row == p_col) & (jr < ic)
    rank = jnp.sum((gt | tie).astype(jnp.int32), axis=1, keepdims=True)
    rr = lax.broadcasted_iota(jnp.int32, (L, K), 1)
    ef = (rank == rr).astype(jnp.float32)
    xsel_o[0] = lax.dot_general(ef, x2, (((0,), (0,)), ((), ())),
                                precision=lax.Precision.HIGHEST)
    psel_o[0] = lax.dot_general(ef, pos[0], (((0,), (0,)), ((), ())),
                                precision=lax.Precision.HIGHEST)


def _row2d(x):
    return x.reshape(1, -1)


def kernel(input, positin_embedding, Wq, bq, Wk, bk, Wv, bv, Wo, bo,
           n1g, n1b, n2g, n2b, f1W, f1b, f2W, f2b,
           p0W, p0b, p1W, p1b, p3W, p3b):
    tok_spec = pl.BlockSpec((1, L, D), lambda b: (b, 0, 0))
    full = lambda a: pl.BlockSpec(a.shape, lambda b: (0,) * a.ndim)
    ws = [Wq, _row2d(bq), Wk, _row2d(bk), Wv, _row2d(bv), Wo, _row2d(bo),
          _row2d(n1g), _row2d(n1b), _row2d(n2g), _row2d(n2b),
          f1W, _row2d(f1b), f2W, _row2d(f2b),
          p0W, _row2d(p0b), p1W, _row2d(p1b), p3W, _row2d(p3b)]
    xsel, psel = pl.pallas_call(
        _enc_body,
        grid=(B,),
        in_specs=[tok_spec, tok_spec] + [full(w) for w in ws],
        out_specs=[pl.BlockSpec((1, K, D), lambda b: (b, 0, 0)),
                   pl.BlockSpec((1, K, D), lambda b: (b, 0, 0))],
        out_shape=[jax.ShapeDtypeStruct((B, K, D), jnp.float32),
                   jax.ShapeDtypeStruct((B, K, D), jnp.float32)],
        compiler_params=pltpu.CompilerParams(
            dimension_semantics=("arbitrary",)),
    )(input, positin_embedding, *ws)
    return (xsel, psel)


# speedup vs baseline: 1.0397x; 1.0397x over previous
import functools
import jax
import jax.numpy as jnp
import numpy as np
from jax import lax
from jax.experimental import pallas as pl
from jax.experimental.pallas import tpu as pltpu, tpu_sc as plsc

B, L, D, HID, PD = 4, 2048, 128, 512, 32
K = L // 2
H = 2
DH = D // H
JC = 1024

_f = np.float32


def _gelu_xla(a):
    one = _f(1.0)
    u = (-a) * _f(0.707106769)
    y = u * u
    t = y * _f(7.85386146e-05)
    t = t + _f(-0.000801019371)
    t = t * y + _f(0.00518832775)
    t = t * y + _f(-0.0268538129)
    t = t * y + _f(0.112835854)
    t = t * y + _f(-0.37612626)
    t = t * y + _f(1.12837911)
    small = one - u * t
    ny = -y
    ez = jnp.exp(ny)
    absu = jnp.abs(u)
    q = one / absu
    ezq = ez * q
    w = one / y
    p = w * _f(0.0232682)
    p = p + _f(-0.138703942)
    p = p * w + _f(0.368742466)
    p = p * w + _f(-0.582473278)
    p = p * w + _f(0.621000469)
    p = p * w + _f(-0.494451523)
    p = p * w + _f(0.340488)
    p = p * w + _f(-0.274112701)
    p = p * w + _f(0.563825965)
    r = w * _f(-10.477664)
    r = r + _f(12.9772)
    r = r * w + _f(-7.49551868)
    r = r * w + _f(2.92101908)
    r = r * w + _f(-1.01526523)
    r = r * w + _f(0.42184633)
    r = r * w + _f(-0.282076746)
    r = r * w + _f(0.564189494)
    poly = jnp.where(absu < _f(2.0), p, r)
    val = ezq * poly
    val = jnp.where(ny < _f(-88.7228394), _f(0.0), val)
    large = jnp.where(u < _f(0.0), _f(2.0) - val, val)
    erfc = jnp.where(absu < one, small, large)
    return (a * _f(0.5)) * erfc


def _mm(a, b):
    return jax.lax.dot_general(a, b, (((1,), (0,)), ((), ())),
                               preferred_element_type=jnp.float32)


def _mm_t(a, b):
    return jax.lax.dot_general(a, b, (((1,), (1,)), ((), ())),
                               preferred_element_type=jnp.float32)


def _mm_seq256(a, b):
    kk = a.shape[1]
    acc = _mm(a[:, 0:256], b[0:256, :])
    for i in range(1, kk // 256):
        acc = acc + _mm(a[:, i * 256:(i + 1) * 256], b[i * 256:(i + 1) * 256, :])
    return acc


def _rowsum_xla(x):
    n = x.shape[1]
    acc = x[:, 0:128]
    for c2 in range(1, n // 128):
        acc = acc + x[:, c2 * 128:(c2 + 1) * 128]
    a8 = acc[:, 0:8]
    for i in range(1, 16):
        a8 = a8 + acc[:, i * 8:(i + 1) * 8]
    a4 = a8[:, 0:4] + a8[:, 4:8]
    a2 = a4[:, 0:2] + a4[:, 2:4]
    return a2[:, 0:1] + a2[:, 1:2]


def _ln_stats(o):
    mean = _rowsum_xla(o) * _f(0.0078125)
    c = o - mean
    var = _rowsum_xla(c * c) * _f(0.0078125)
    sq = jnp.sqrt(var + _f(1e-5))
    return mean, sq


def _enc_body(inp, pos, Wq, bq, Wk, bk, Wv, bv, Wo, bo, n1g, n1b, n2g, n2b,
              f1W, f1b, f2W, f2b, p0W, p0b, p1W, p1b, p3W, p3b,
              x2_o, ids_o):
    x0 = inp[0] + pos[0]
    bf = jnp.bfloat16
    qr = (_mm(x0, Wq[...]) + bq[...]).astype(bf).astype(jnp.float32)
    kr = (_mm(x0, Wk[...]) + bk[...]).astype(bf).astype(jnp.float32)
    v32 = _mm(x0, Wv[...]) + bv[...]
    outs = []
    for h in range(H):
        qh = qr[:, h * DH:(h + 1) * DH]
        kh = kr[:, h * DH:(h + 1) * DH]
        vh = v32[:, h * DH:(h + 1) * DH]
        t0 = _mm_t(qh, kh[0:JC, :]) * _f(0.125)
        m1 = jnp.max(t0, axis=1, keepdims=True)
        e0 = jnp.exp(t0 - m1)
        l1 = jnp.sum(e0, axis=1, keepdims=True)
        r0 = _mm_seq256(e0, vh[0:JC, :])
        o1 = r0 * (_f(1.0) / l1)
        t1 = _mm_t(qh, kh[JC:L, :]) * _f(0.125)
        mt = jnp.max(t1, axis=1, keepdims=True)
        m2 = jnp.maximum(m1, mt)
        corr = jnp.where(m1 == m2, _f(0.0), m1 - m2)
        c = jnp.exp(corr) * l1
        e1 = jnp.exp(t1 - m2)
        l2 = c + jnp.sum(e1, axis=1, keepdims=True)
        r1 = c * o1
        for i2 in range(JC // 256):
            r1 = r1 + _mm(e1[:, i2 * 256:(i2 + 1) * 256],
                          vh[JC + i2 * 256:JC + (i2 + 1) * 256, :])
        outs.append(r1 * (_f(1.0) / l2))
    o = jnp.concatenate(outs, axis=1)
    obf = o.astype(bf).astype(jnp.float32)
    oc = _mm(obf, Wo[...])
    ln1_in = oc + bo[...]
    mean1, sq1 = _ln_stats(ln1_in)
    ln1 = ((ln1_in - mean1) / sq1) * n1g[...] + n1b[...]
    x1 = x0 + ln1
    a1 = _mm(x1, f1W[...]) + f1b[...]
    g1 = _gelu_xla(a1)
    ln2_in = _mm(g1, f2W[...]) + f2b[...]
    mean2, sq2 = _ln_stats(ln2_in)
    ln2 = ((ln2_in - mean2) / sq2) * n2g[...] + n2b[...]
    x2 = (x0 + ln2) + ln1
    a_s0 = _mm(x2, p0W[...]) + p0b[...]
    g_s0 = _gelu_xla(a_s0)
    a_s1 = _mm(g_s0, p1W[...]) + p1b[...]
    g_s1 = _gelu_xla(a_s1)
    s_row = lax.dot_general(p3W[...], g_s1, (((0,), (1,)), ((), ())),
                            preferred_element_type=jnp.float32)
    sb_row = s_row + p3b[...]
    m_s = jnp.max(sb_row, axis=1, keepdims=True)
    e_row = jnp.exp(sb_row - m_s)
    z = _rowsum_xla(e_row)
    p_row = e_row / z
    ic = lax.broadcasted_iota(jnp.int32, (L, 1), 0)
    jr = lax.broadcasted_iota(jnp.int32, (1, L), 1)
    eye = (ic == jr).astype(jnp.float32)
    p_col = lax.dot_general(eye, p_row, (((1,), (1,)), ((), ())),
                            precision=lax.Precision.HIGHEST)
    gt = p_row > p_col
    tie = (p_row == p_col) & (jr < ic)
    rank = jnp.sum((gt | tie).astype(jnp.int32), axis=1, keepdims=True)
    rr = lax.broadcasted_iota(jnp.int32, (L, K), 1)
    sel = rank == rr
    iota_i = lax.broadcasted_iota(jnp.int32, (L, K), 0)
    ids = jnp.sum(jnp.where(sel, iota_i, 0), axis=0, keepdims=True)
    b_id = pl.program_id(0)
    ids_o[0] = ids + b_id * L
    x2_o[0] = x2


NC, NS = 2, 16
NW = NC * NS
PER_W = (B * K) // NW

_sc_mesh = plsc.VectorSubcoreMesh(core_axis_name="c", subcore_axis_name="s")


@functools.partial(
    pl.kernel, mesh=_sc_mesh,
    out_type=[jax.ShapeDtypeStruct((B * K, D), jnp.float32),
              jax.ShapeDtypeStruct((B * K, D), jnp.float32)],
    scratch_types=[
        pltpu.VMEM((PER_W,), jnp.int32),
        pltpu.VMEM((PER_W, D), jnp.float32),
        pltpu.VMEM((PER_W, D), jnp.float32),
        pltpu.SemaphoreType.DMA,
        pltpu.SemaphoreType.DMA,
    ],
)
def _sc_gather(x2_hbm, pos_hbm, idx_hbm, xo_hbm, po_hbm,
               idx_v, xrows, prows, s1, s2):
    wid = lax.axis_index("s") * NC + lax.axis_index("c")
    base = wid * PER_W
    pltpu.sync_copy(idx_hbm.at[pl.ds(base, PER_W)], idx_v)
    a = pltpu.async_copy(x2_hbm.at[idx_v], xrows, s1)
    b = pltpu.async_copy(pos_hbm.at[idx_v], prows, s2)
    a.wait()
    b.wait()
    pltpu.sync_copy(xrows, xo_hbm.at[pl.ds(base, PER_W)])
    pltpu.sync_copy(prows, po_hbm.at[pl.ds(base, PER_W)])


def _row2d(x):
    return x.reshape(1, -1)


def kernel(input, positin_embedding, Wq, bq, Wk, bk, Wv, bv, Wo, bo,
           n1g, n1b, n2g, n2b, f1W, f1b, f2W, f2b,
           p0W, p0b, p1W, p1b, p3W, p3b):
    tok_spec = pl.BlockSpec((1, L, D), lambda b: (b, 0, 0))
    full = lambda a: pl.BlockSpec(a.shape, lambda b: (0,) * a.ndim)
    ws = [Wq, _row2d(bq), Wk, _row2d(bk), Wv, _row2d(bv), Wo, _row2d(bo),
          _row2d(n1g), _row2d(n1b), _row2d(n2g), _row2d(n2b),
          f1W, _row2d(f1b), f2W, _row2d(f2b),
          p0W, _row2d(p0b), p1W, _row2d(p1b), p3W, _row2d(p3b)]
    x2f, ids = pl.pallas_call(
        _enc_body,
        grid=(B,),
        in_specs=[tok_spec, tok_spec] + [full(w) for w in ws],
        out_specs=[pl.BlockSpec((1, L, D), lambda b: (b, 0, 0)),
                   pl.BlockSpec((1, 1, K), lambda b: (b, 0, 0))],
        out_shape=[jax.ShapeDtypeStruct((B, L, D), jnp.float32),
                   jax.ShapeDtypeStruct((B, 1, K), jnp.int32)],
        compiler_params=pltpu.CompilerParams(
            dimension_semantics=("arbitrary",)),
    )(input, positin_embedding, *ws)
    xs, ps = _sc_gather(x2f.reshape(B * L, D),
                        positin_embedding.reshape(B * L, D),
                        ids.reshape(B * K))
    return (xs.reshape(B, K, D), ps.reshape(B, K, D))
